# baseline (speedup 1.0000x reference)
import functools

import jax
import jax.numpy as jnp
from jax import lax
from jax.experimental import pallas as pl
from jax.experimental.pallas import tpu as pltpu
from jax.experimental.pallas import tpu_sc as plsc

N = 10000
E = 320000
D = 128

NC = 2
NS = 16
NW = NC * NS
CHUNK = 128
NCHUNK = 80
EPT = NCHUNK * CHUNK
EPAD = NW * EPT
NA = N + 8
ROWS_A = 632
ROWS_LAST = NA - (NS - 1) * ROWS_A

_mesh = plsc.VectorSubcoreMesh(core_axis_name="c", subcore_axis_name="s")


def _striped(s, copy_fn):
  row0 = pl.multiple_of(s * ROWS_A, 8)

  @pl.when(s < NS - 1)
  def _():
    copy_fn(row0, ROWS_A)

  @pl.when(s == NS - 1)
  def _():
    copy_fn(row0, ROWS_LAST)


SB = 40


@functools.partial(
    pl.kernel,
    out_type=jax.ShapeDtypeStruct((NC, NA, D), jnp.float32),
    mesh=_mesh,
    scratch_types=(
        pltpu.VMEM((SB, CHUNK), jnp.int32),
        pltpu.VMEM((SB, CHUNK), jnp.int32),
        pltpu.VMEM((CHUNK, D), jnp.float32),
        pltpu.VMEM((CHUNK, D), jnp.float32),
        pltpu.VMEM_SHARED((NA, D), jnp.float32),
        pltpu.SemaphoreType.DMA,
        pltpu.SemaphoreType.DMA,
    ),
)
def _sc_agg(h_hbm, src_hbm, dst_hbm, z_rows, sums_out,
            idx_s, idx_d, rows0, rows1, sums_sh, sem0, sem1):
  c = lax.axis_index("c")
  s = lax.axis_index("s")
  wid = s * NC + c

  _striped(s, lambda r, n: pltpu.sync_copy(z_rows.at[pl.ds(r, n)],
                                           sums_sh.at[pl.ds(r, n)]))
  plsc.subcore_barrier()

  rows = (rows0, rows1)
  sems = (sem0, sem1)

  for sb in range(NCHUNK // SB):
    pltpu.sync_copy(src_hbm.at[wid, pl.ds(sb * SB, SB)], idx_s)
    pltpu.sync_copy(dst_hbm.at[wid, pl.ds(sb * SB, SB)], idx_d)
    pltpu.async_copy(h_hbm.at[idx_s.at[0]], rows0, sem0)
    pltpu.async_copy(h_hbm.at[idx_s.at[1]], rows1, sem1)

    def body(j, carry):
      for b in range(2):
        ch = 2 * j + b
        pltpu.make_async_copy(h_hbm.at[idx_s.at[ch]], rows[b], sems[b]).wait()
        pltpu.sync_copy(rows[b], sums_sh.at[idx_d.at[ch]], add=True)
        pltpu.async_copy(h_hbm.at[idx_s.at[ch + 2]], rows[b], sems[b])
      return carry

    lax.fori_loop(0, SB // 2 - 1, body, 0)
    for b in range(2):
      ch = SB - 2 + b
      pltpu.make_async_copy(h_hbm.at[idx_s.at[ch]], rows[b], sems[b]).wait()
      pltpu.sync_copy(rows[b], sums_sh.at[idx_d.at[ch]], add=True)

  plsc.subcore_barrier()

  _striped(s, lambda r, n: pltpu.sync_copy(sums_sh.at[pl.ds(r, n)],
                                           sums_out.at[c, pl.ds(r, n)]))


@functools.partial(
    pl.kernel,
    out_type=jax.ShapeDtypeStruct((NC, NA, D), jnp.float32),
    mesh=_mesh,
    scratch_types=(
        pltpu.VMEM((NCHUNK, CHUNK), jnp.int32),
        pltpu.VMEM((CHUNK, D), jnp.float32),
        pltpu.VMEM_SHARED((NA, D), jnp.float32),
    ),
)
def _sc_deg(dst_hbm, z_rows, ones_hbm, deg_out, idx_d, ones_v, deg_sh):
  c = lax.axis_index("c")
  s = lax.axis_index("s")
  wid = s * NC + c

  _striped(s, lambda r, n: pltpu.sync_copy(z_rows.at[pl.ds(r, n)],
                                           deg_sh.at[pl.ds(r, n)]))
  pltpu.sync_copy(ones_hbm, ones_v)
  pltpu.sync_copy(dst_hbm.at[wid], idx_d)
  plsc.subcore_barrier()

  def body(j, carry):
    pltpu.sync_copy(ones_v, deg_sh.at[idx_d.at[j]], add=True)
    return carry

  lax.fori_loop(0, NCHUNK, body, 0)
  plsc.subcore_barrier()

  _striped(s, lambda r, n: pltpu.sync_copy(deg_sh.at[pl.ds(r, n)],
                                           deg_out.at[c, pl.ds(r, n)]))


RB = 2000


def _dense_body(x_ref, sp_ref, dp_ref, w12_ref, w2t_ref, b12_ref,
                gamma_ref, beta_ref, out_ref, *, with_ln):
  x = x_ref[...]
  ssum = sp_ref[0] + sp_ref[1]
  deg = dp_ref[0, :, 0:1] + dp_ref[1, :, 0:1]
  mean = ssum / jnp.maximum(deg, 1.0)
  out = (jnp.dot(x, w12_ref[...], preferred_element_type=jnp.float32)
         + b12_ref[...]
         - jnp.dot(mean, w2t_ref[...], preferred_element_type=jnp.float32))
  out = jnp.where(deg > 0.0, out, x)
  if with_ln:
    h = jnp.maximum(out, 0.0)
    mu = jnp.mean(h, axis=1, keepdims=True)
    var = jnp.mean((h - mu) * (h - mu), axis=1, keepdims=True)
    out = (h - mu) * lax.rsqrt(var + 1e-5) * gamma_ref[...] + beta_ref[...]
  out_ref[...] = out


def _make_dense(with_ln):
  body = functools.partial(_dense_body, with_ln=with_ln)
  return pl.pallas_call(
      body,
      grid=(N // RB,),
      in_specs=[
          pl.BlockSpec((RB, D), lambda i: (i, 0)),
          pl.BlockSpec((NC, RB, D), lambda i: (0, i, 0)),
          pl.BlockSpec((NC, RB, D), lambda i: (0, i, 0)),
          pl.BlockSpec((D, D), lambda i: (0, 0)),
          pl.BlockSpec((D, D), lambda i: (0, 0)),
          pl.BlockSpec((1, D), lambda i: (0, 0)),
          pl.BlockSpec((1, D), lambda i: (0, 0)),
          pl.BlockSpec((1, D), lambda i: (0, 0)),
      ],
      out_specs=pl.BlockSpec((RB, D), lambda i: (i, 0)),
      out_shape=jax.ShapeDtypeStruct((N, D), jnp.float32),
  )


_dense_ln = _make_dense(True)
_dense_out = _make_dense(False)


def kernel(x, edge_index, W1_0, b1_0, W2_0, b2_0, gamma, beta,
           W1_1, b1_1, W2_1, b2_1):
  src = edge_index[0].astype(jnp.int32)
  dst = edge_index[1].astype(jnp.int32)
  src = jnp.concatenate([src, jnp.zeros((EPAD - E,), jnp.int32)])
  dst = jnp.concatenate([dst, jnp.full((EPAD - E,), N, jnp.int32)])
  src3 = src.reshape(NW, NCHUNK, CHUNK)
  dst3 = dst.reshape(NW, NCHUNK, CHUNK)

  z_rows = jnp.zeros((NA, D), jnp.float32)
  ones = jnp.ones((CHUNK, D), jnp.float32)

  w12_0 = (W1_0 + W2_0).T
  w2t_0 = W2_0.T
  b12_0 = (b1_0 + b2_0).reshape(1, D)
  w12_1 = (W1_1 + W2_1).T
  w2t_1 = W2_1.T
  b12_1 = (b1_1 + b2_1).reshape(1, D)
  gamma2 = gamma.reshape(1, D)
  beta2 = beta.reshape(1, D)

  degp = _sc_deg(dst3, z_rows, ones)
  sums0 = _sc_agg(x, src3, dst3, z_rows)
  h1 = _dense_ln(x, sums0, degp, w12_0, w2t_0, b12_0, gamma2, beta2)
  sums1 = _sc_agg(h1, src3, dst3, z_rows)
  out = _dense_out(h1, sums1, degp, w12_1, w2t_1, b12_1, gamma2, beta2)
  return out


# speedup vs baseline: 2.7425x; 2.7425x over previous
import functools

import jax
import jax.numpy as jnp
from jax import lax
from jax.experimental import pallas as pl
from jax.experimental.pallas import tpu as pltpu
from jax.experimental.pallas import tpu_sc as plsc

N = 10000
E = 320000
D = 128

NC = 2
NS = 16
NW = NC * NS
CHUNK = 80
NCHUNK = 125
KB = 25
NB = NCHUNK // KB
EPT = NCHUNK * CHUNK
EPAD = NW * EPT
NA = N
ROWS_A = 632
ROWS_LAST = NA - (NS - 1) * ROWS_A

_mesh = plsc.VectorSubcoreMesh(core_axis_name="c", subcore_axis_name="s")


def _striped(s, copy_fn):
  row0 = pl.multiple_of(s * ROWS_A, 8)

  @pl.when(s < NS - 1)
  def _():
    copy_fn(row0, ROWS_A)

  @pl.when(s == NS - 1)
  def _():
    copy_fn(row0, ROWS_LAST)


@functools.partial(
    pl.kernel,
    out_type=jax.ShapeDtypeStruct((NC, NA, D), jnp.float32),
    mesh=_mesh,
    scratch_types=(
        pltpu.VMEM((KB, CHUNK), jnp.int32),
        pltpu.VMEM((KB, CHUNK), jnp.int32),
        pltpu.VMEM((CHUNK, D), jnp.float32),
        pltpu.VMEM((CHUNK, D), jnp.float32),
        pltpu.VMEM_SHARED((NA, D), jnp.float32),
        pltpu.SemaphoreType.DMA,
        pltpu.SemaphoreType.DMA,
    ),
)
def _sc_agg(h_hbm, src_hbm, dst_hbm, z_rows, sums_out,
            idx_s, idx_d, rows0, rows1, sums_sh, sem0, sem1):
  c = lax.axis_index("c")
  s = lax.axis_index("s")
  wid = s * NC + c

  _striped(s, lambda r, n: pltpu.sync_copy(z_rows.at[pl.ds(r, n)],
                                           sums_sh.at[pl.ds(r, n)]))
  plsc.subcore_barrier()

  rows = (rows0, rows1)
  sems = (sem0, sem1)

  for blk in range(NB):
    pltpu.sync_copy(src_hbm.at[wid, blk], idx_s)
    pltpu.sync_copy(dst_hbm.at[wid, blk], idx_d)

    pltpu.async_copy(h_hbm.at[idx_s.at[0]], rows0, sem0)
    pltpu.async_copy(h_hbm.at[idx_s.at[1]], rows1, sem1)

    def body(j, carry):
      for b in range(2):
        ch = 2 * j + b
        pltpu.make_async_copy(h_hbm.at[idx_s.at[ch]], rows[b], sems[b]).wait()
        pltpu.sync_copy(rows[b], sums_sh.at[idx_d.at[ch]], add=True)

        @pl.when(ch + 2 < KB)
        def _():
          pltpu.async_copy(h_hbm.at[idx_s.at[ch + 2]], rows[b], sems[b])
      return carry

    lax.fori_loop(0, KB // 2, body, 0)
    ch = KB - 1
    pltpu.make_async_copy(h_hbm.at[idx_s.at[ch]], rows[ch % 2],
                          sems[ch % 2]).wait()
    pltpu.sync_copy(rows[ch % 2], sums_sh.at[idx_d.at[ch]], add=True)

  plsc.subcore_barrier()

  _striped(s, lambda r, n: pltpu.sync_copy(sums_sh.at[pl.ds(r, n)],
                                           sums_out.at[c, pl.ds(r, n)]))


@functools.partial(
    pl.kernel,
    out_type=jax.ShapeDtypeStruct((NC, NA, D), jnp.float32),
    mesh=_mesh,
    scratch_types=(
        pltpu.VMEM((NCHUNK, CHUNK), jnp.int32),
        pltpu.VMEM((CHUNK, D), jnp.float32),
        pltpu.VMEM_SHARED((NA, D), jnp.float32),
    ),
)
def _sc_deg(dst_hbm, z_rows, ones_hbm, deg_out, idx_d, ones_v, deg_sh):
  c = lax.axis_index("c")
  s = lax.axis_index("s")
  wid = s * NC + c

  _striped(s, lambda r, n: pltpu.sync_copy(z_rows.at[pl.ds(r, n)],
                                           deg_sh.at[pl.ds(r, n)]))
  pltpu.sync_copy(ones_hbm, ones_v)
  pltpu.sync_copy(dst_hbm.at[wid], idx_d)
  plsc.subcore_barrier()

  def body(j, carry):
    pltpu.sync_copy(ones_v, deg_sh.at[idx_d.at[j]], add=True)
    return carry

  lax.fori_loop(0, NCHUNK, body, 0)
  plsc.subcore_barrier()

  _striped(s, lambda r, n: pltpu.sync_copy(deg_sh.at[pl.ds(r, n)],
                                           deg_out.at[c, pl.ds(r, n)]))


RB = 2000


def _dense_body(x_ref, sp_ref, dp_ref, w12_ref, w2t_ref, b12_ref,
                gamma_ref, beta_ref, out_ref, *, with_ln):
  x = x_ref[...]
  ssum = sp_ref[0] + sp_ref[1]
  deg = dp_ref[0, :, 0:1] + dp_ref[1, :, 0:1]
  mean = ssum / jnp.maximum(deg, 1.0)
  out = (jnp.dot(x, w12_ref[...], preferred_element_type=jnp.float32)
         + b12_ref[...]
         - jnp.dot(mean, w2t_ref[...], preferred_element_type=jnp.float32))
  out = jnp.where(deg > 0.0, out, x)
  if with_ln:
    h = jnp.maximum(out, 0.0)
    mu = jnp.mean(h, axis=1, keepdims=True)
    var = jnp.mean((h - mu) * (h - mu), axis=1, keepdims=True)
    out = (h - mu) * lax.rsqrt(var + 1e-5) * gamma_ref[...] + beta_ref[...]
  out_ref[...] = out


def _make_dense(with_ln):
  body = functools.partial(_dense_body, with_ln=with_ln)
  return pl.pallas_call(
      body,
      grid=(N // RB,),
      in_specs=[
          pl.BlockSpec((RB, D), lambda i: (i, 0)),
          pl.BlockSpec((NC, RB, D), lambda i: (0, i, 0)),
          pl.BlockSpec((NC, RB, D), lambda i: (0, i, 0)),
          pl.BlockSpec((D, D), lambda i: (0, 0)),
          pl.BlockSpec((D, D), lambda i: (0, 0)),
          pl.BlockSpec((1, D), lambda i: (0, 0)),
          pl.BlockSpec((1, D), lambda i: (0, 0)),
          pl.BlockSpec((1, D), lambda i: (0, 0)),
      ],
      out_specs=pl.BlockSpec((RB, D), lambda i: (i, 0)),
      out_shape=jax.ShapeDtypeStruct((N, D), jnp.float32),
  )


_dense_ln = _make_dense(True)
_dense_out = _make_dense(False)


def kernel(x, edge_index, W1_0, b1_0, W2_0, b2_0, gamma, beta,
           W1_1, b1_1, W2_1, b2_1):
  src = edge_index[0].astype(jnp.int32)
  dst = edge_index[1].astype(jnp.int32)
  src = jnp.concatenate([src, jnp.zeros((EPAD - E,), jnp.int32)])
  dst = jnp.concatenate([dst, jnp.full((EPAD - E,), N, jnp.int32)])
  src4 = src.reshape(NW, NB, KB, CHUNK)
  dst4 = dst.reshape(NW, NB, KB, CHUNK)
  dst3 = dst.reshape(NW, NCHUNK, CHUNK)

  z_rows = jnp.zeros((NA, D), jnp.float32)
  ones = jnp.ones((CHUNK, D), jnp.float32)

  w12_0 = (W1_0 + W2_0).T
  w2t_0 = W2_0.T
  b12_0 = (b1_0 + b2_0).reshape(1, D)
  w12_1 = (W1_1 + W2_1).T
  w2t_1 = W2_1.T
  b12_1 = (b1_1 + b2_1).reshape(1, D)
  gamma2 = gamma.reshape(1, D)
  beta2 = beta.reshape(1, D)

  degp = _sc_deg(dst3, z_rows, ones)
  sums0 = _sc_agg(x, src4, dst4, z_rows)
  h1 = _dense_ln(x, sums0, degp, w12_0, w2t_0, b12_0, gamma2, beta2)
  sums1 = _sc_agg(h1, src4, dst4, z_rows)
  out = _dense_out(h1, sums1, degp, w12_1, w2t_1, b12_1, gamma2, beta2)
  return out


# speedup vs baseline: 3.0349x; 1.1066x over previous
import functools

import jax
import jax.numpy as jnp
from jax import lax
from jax.experimental import pallas as pl
from jax.experimental.pallas import tpu as pltpu
from jax.experimental.pallas import tpu_sc as plsc

N = 10000
E = 320000
D = 128

NC = 2
NS = 16
NW = NC * NS
CHUNK = 80
NCHUNK = 125
KB = 25
NB = NCHUNK // KB
EPT = NCHUNK * CHUNK
EPAD = NW * EPT
NA = N
ROWS_A = 632
ROWS_LAST = NA - (NS - 1) * ROWS_A

_mesh = plsc.VectorSubcoreMesh(core_axis_name="c", subcore_axis_name="s")


def _striped(s, copy_fn):
  row0 = pl.multiple_of(s * ROWS_A, 8)

  @pl.when(s < NS - 1)
  def _():
    copy_fn(row0, ROWS_A)

  @pl.when(s == NS - 1)
  def _():
    copy_fn(row0, ROWS_LAST)


@functools.partial(
    pl.kernel,
    out_type=jax.ShapeDtypeStruct((NC, NA, D), jnp.float32),
    mesh=_mesh,
    scratch_types=(
        pltpu.VMEM((KB, CHUNK), jnp.int32),
        pltpu.VMEM((KB, CHUNK), jnp.int32),
        pltpu.VMEM((CHUNK, D), jnp.float32),
        pltpu.VMEM((CHUNK, D), jnp.float32),
        pltpu.VMEM((CHUNK, D), jnp.float32),
        pltpu.VMEM_SHARED((NA, D), jnp.float32),
        pltpu.SemaphoreType.DMA,
        pltpu.SemaphoreType.DMA,
        pltpu.SemaphoreType.DMA,
    ),
)
def _sc_agg(h_hbm, src_hbm, dst_hbm, z_rows, sums_out,
            idx_s, idx_d, rows0, rows1, rows2, sums_sh, sem0, sem1, sem2):
  c = lax.axis_index("c")
  s = lax.axis_index("s")
  wid = s * NC + c

  _striped(s, lambda r, n: pltpu.sync_copy(z_rows.at[pl.ds(r, n)],
                                           sums_sh.at[pl.ds(r, n)]))
  plsc.subcore_barrier()

  rows = (rows0, rows1, rows2)
  sems = (sem0, sem1, sem2)
  NBUF = len(rows)

  for blk in range(NB):
    pltpu.sync_copy(src_hbm.at[wid, blk], idx_s)
    pltpu.sync_copy(dst_hbm.at[wid, blk], idx_d)

    for b in range(NBUF):
      pltpu.async_copy(h_hbm.at[idx_s.at[b]], rows[b], sems[b])

    def body(j, carry):
      for b in range(NBUF):
        ch = NBUF * j + b
        pltpu.make_async_copy(h_hbm.at[idx_s.at[ch]], rows[b], sems[b]).wait()
        pltpu.sync_copy(rows[b], sums_sh.at[idx_d.at[ch]], add=True)

        @pl.when(ch + NBUF < KB)
        def _():
          pltpu.async_copy(h_hbm.at[idx_s.at[ch + NBUF]], rows[b], sems[b])
      return carry

    lax.fori_loop(0, KB // NBUF, body, 0)
    for ch in range((KB // NBUF) * NBUF, KB):
      pltpu.make_async_copy(h_hbm.at[idx_s.at[ch]], rows[ch % NBUF],
                            sems[ch % NBUF]).wait()
      pltpu.sync_copy(rows[ch % NBUF], sums_sh.at[idx_d.at[ch]], add=True)

  plsc.subcore_barrier()

  _striped(s, lambda r, n: pltpu.sync_copy(sums_sh.at[pl.ds(r, n)],
                                           sums_out.at[c, pl.ds(r, n)]))


@functools.partial(
    pl.kernel,
    out_type=jax.ShapeDtypeStruct((NC, NA, D), jnp.float32),
    mesh=_mesh,
    scratch_types=(
        pltpu.VMEM((NCHUNK, CHUNK), jnp.int32),
        pltpu.VMEM((CHUNK, D), jnp.float32),
        pltpu.VMEM_SHARED((NA, D), jnp.float32),
    ),
)
def _sc_deg(dst_hbm, z_rows, ones_hbm, deg_out, idx_d, ones_v, deg_sh):
  c = lax.axis_index("c")
  s = lax.axis_index("s")
  wid = s * NC + c

  _striped(s, lambda r, n: pltpu.sync_copy(z_rows.at[pl.ds(r, n)],
                                           deg_sh.at[pl.ds(r, n)]))
  pltpu.sync_copy(ones_hbm, ones_v)
  pltpu.sync_copy(dst_hbm.at[wid], idx_d)
  plsc.subcore_barrier()

  def body(j, carry):
    pltpu.sync_copy(ones_v, deg_sh.at[idx_d.at[j]], add=True)
    return carry

  lax.fori_loop(0, NCHUNK, body, 0)
  plsc.subcore_barrier()

  _striped(s, lambda r, n: pltpu.sync_copy(deg_sh.at[pl.ds(r, n)],
                                           deg_out.at[c, pl.ds(r, n)]))


RB = 2000


def _dense_body(x_ref, sp_ref, dp_ref, w12_ref, w2t_ref, b12_ref,
                gamma_ref, beta_ref, out_ref, *, with_ln):
  x = x_ref[...]
  ssum = sp_ref[0] + sp_ref[1]
  deg = dp_ref[0, :, 0:1] + dp_ref[1, :, 0:1]
  mean = ssum / jnp.maximum(deg, 1.0)
  out = (jnp.dot(x, w12_ref[...], preferred_element_type=jnp.float32)
         + b12_ref[...]
         - jnp.dot(mean, w2t_ref[...], preferred_element_type=jnp.float32))
  out = jnp.where(deg > 0.0, out, x)
  if with_ln:
    h = jnp.maximum(out, 0.0)
    mu = jnp.mean(h, axis=1, keepdims=True)
    var = jnp.mean((h - mu) * (h - mu), axis=1, keepdims=True)
    out = (h - mu) * lax.rsqrt(var + 1e-5) * gamma_ref[...] + beta_ref[...]
  out_ref[...] = out


def _make_dense(with_ln):
  body = functools.partial(_dense_body, with_ln=with_ln)
  return pl.pallas_call(
      body,
      grid=(N // RB,),
      in_specs=[
          pl.BlockSpec((RB, D), lambda i: (i, 0)),
          pl.BlockSpec((NC, RB, D), lambda i: (0, i, 0)),
          pl.BlockSpec((NC, RB, D), lambda i: (0, i, 0)),
          pl.BlockSpec((D, D), lambda i: (0, 0)),
          pl.BlockSpec((D, D), lambda i: (0, 0)),
          pl.BlockSpec((1, D), lambda i: (0, 0)),
          pl.BlockSpec((1, D), lambda i: (0, 0)),
          pl.BlockSpec((1, D), lambda i: (0, 0)),
      ],
      out_specs=pl.BlockSpec((RB, D), lambda i: (i, 0)),
      out_shape=jax.ShapeDtypeStruct((N, D), jnp.float32),
  )


_dense_ln = _make_dense(True)
_dense_out = _make_dense(False)


def kernel(x, edge_index, W1_0, b1_0, W2_0, b2_0, gamma, beta,
           W1_1, b1_1, W2_1, b2_1):
  src = edge_index[0].astype(jnp.int32)
  dst = edge_index[1].astype(jnp.int32)
  src = jnp.concatenate([src, jnp.zeros((EPAD - E,), jnp.int32)])
  dst = jnp.concatenate([dst, jnp.full((EPAD - E,), N, jnp.int32)])
  src4 = src.reshape(NW, NB, KB, CHUNK)
  dst4 = dst.reshape(NW, NB, KB, CHUNK)
  dst3 = dst.reshape(NW, NCHUNK, CHUNK)

  z_rows = jnp.zeros((NA, D), jnp.float32)
  ones = jnp.ones((CHUNK, D), jnp.float32)

  w12_0 = (W1_0 + W2_0).T
  w2t_0 = W2_0.T
  b12_0 = (b1_0 + b2_0).reshape(1, D)
  w12_1 = (W1_1 + W2_1).T
  w2t_1 = W2_1.T
  b12_1 = (b1_1 + b2_1).reshape(1, D)
  gamma2 = gamma.reshape(1, D)
  beta2 = beta.reshape(1, D)

  degp = _sc_deg(dst3, z_rows, ones)
  sums0 = _sc_agg(x, src4, dst4, z_rows)
  h1 = _dense_ln(x, sums0, degp, w12_0, w2t_0, b12_0, gamma2, beta2)
  sums1 = _sc_agg(h1, src4, dst4, z_rows)
  out = _dense_out(h1, sums1, degp, w12_1, w2t_1, b12_1, gamma2, beta2)
  return out


# speedup vs baseline: 3.0660x; 1.0103x over previous
import functools

import jax
import jax.numpy as jnp
from jax import lax
from jax.experimental import pallas as pl
from jax.experimental.pallas import tpu as pltpu
from jax.experimental.pallas import tpu_sc as plsc

N = 10000
E = 320000
D = 128

NC = 2
NS = 16
NW = NC * NS
CHUNK = 80
NCHUNK = 125
KB = 25
NB = NCHUNK // KB
EPT = NCHUNK * CHUNK
EPAD = NW * EPT
NA = N
ROWS_A = 632
ROWS_LAST = NA - (NS - 1) * ROWS_A

_mesh = plsc.VectorSubcoreMesh(core_axis_name="c", subcore_axis_name="s")


def _striped(s, copy_fn):
  row0 = pl.multiple_of(s * ROWS_A, 8)

  @pl.when(s < NS - 1)
  def _():
    copy_fn(row0, ROWS_A)

  @pl.when(s == NS - 1)
  def _():
    copy_fn(row0, ROWS_LAST)


@functools.partial(
    pl.kernel,
    out_type=jax.ShapeDtypeStruct((NC, NA, D), jnp.float32),
    mesh=_mesh,
    scratch_types=(
        pltpu.VMEM((KB, CHUNK), jnp.int32),
        pltpu.VMEM((KB, CHUNK), jnp.int32),
        pltpu.VMEM((CHUNK, D), jnp.float32),
        pltpu.VMEM((CHUNK, D), jnp.float32),
        pltpu.VMEM((CHUNK, D), jnp.float32),
        pltpu.VMEM_SHARED((NA, D), jnp.float32),
        pltpu.SemaphoreType.DMA,
        pltpu.SemaphoreType.DMA,
        pltpu.SemaphoreType.DMA,
    ),
)
def _sc_agg(h_hbm, src_hbm, dst_hbm, z_rows, sums_out,
            idx_s, idx_d, rows0, rows1, rows2, sums_sh, sem0, sem1, sem2):
  c = lax.axis_index("c")
  s = lax.axis_index("s")
  wid = s * NC + c

  _striped(s, lambda r, n: pltpu.sync_copy(z_rows.at[pl.ds(r, n)],
                                           sums_sh.at[pl.ds(r, n)]))
  plsc.subcore_barrier()

  rows = (rows0, rows1, rows2)
  sems = (sem0, sem1, sem2)
  NBUF = len(rows)

  for blk in range(NB):
    pltpu.sync_copy(src_hbm.at[wid, blk], idx_s)
    pltpu.sync_copy(dst_hbm.at[wid, blk], idx_d)

    for b in range(NBUF):
      pltpu.async_copy(h_hbm.at[idx_s.at[b]], rows[b], sems[b])

    def body(j, carry):
      for b in range(NBUF):
        ch = NBUF * j + b
        pltpu.make_async_copy(h_hbm.at[idx_s.at[ch]], rows[b], sems[b]).wait()
        pltpu.sync_copy(rows[b], sums_sh.at[idx_d.at[ch]], add=True)

        @pl.when(ch + NBUF < KB)
        def _():
          pltpu.async_copy(h_hbm.at[idx_s.at[ch + NBUF]], rows[b], sems[b])
      return carry

    lax.fori_loop(0, KB // NBUF, body, 0)
    for ch in range((KB // NBUF) * NBUF, KB):
      pltpu.make_async_copy(h_hbm.at[idx_s.at[ch]], rows[ch % NBUF],
                            sems[ch % NBUF]).wait()
      pltpu.sync_copy(rows[ch % NBUF], sums_sh.at[idx_d.at[ch]], add=True)

  plsc.subcore_barrier()

  _striped(s, lambda r, n: pltpu.sync_copy(sums_sh.at[pl.ds(r, n)],
                                           sums_out.at[c, pl.ds(r, n)]))


@functools.partial(
    pl.kernel,
    out_type=(jax.ShapeDtypeStruct((NC, NA, D), jnp.float32),
              jax.ShapeDtypeStruct((NC, NA, D), jnp.float32)),
    mesh=_mesh,
    scratch_types=(
        pltpu.VMEM((KB, CHUNK), jnp.int32),
        pltpu.VMEM((KB, CHUNK), jnp.int32),
        pltpu.VMEM((CHUNK, D), jnp.float32),
        pltpu.VMEM((CHUNK, D), jnp.float32),
        pltpu.VMEM((CHUNK, D), jnp.float32),
        pltpu.VMEM_SHARED((NA, D), jnp.float32),
        pltpu.SemaphoreType.DMA,
        pltpu.SemaphoreType.DMA,
        pltpu.SemaphoreType.DMA,
    ),
)
def _sc_agg_deg(h_hbm, src_hbm, dst_hbm, z_rows, ones_hbm, sums_out, deg_out,
                idx_s, idx_d, rows0, rows1, rows2, acc_sh, sem0, sem1, sem2):
  c = lax.axis_index("c")
  s = lax.axis_index("s")
  wid = s * NC + c

  _striped(s, lambda r, n: pltpu.sync_copy(z_rows.at[pl.ds(r, n)],
                                           acc_sh.at[pl.ds(r, n)]))
  plsc.subcore_barrier()

  rows = (rows0, rows1, rows2)
  sems = (sem0, sem1, sem2)
  NBUF = len(rows)

  for blk in range(NB):
    pltpu.sync_copy(src_hbm.at[wid, blk], idx_s)
    pltpu.sync_copy(dst_hbm.at[wid, blk], idx_d)

    for b in range(NBUF):
      pltpu.async_copy(h_hbm.at[idx_s.at[b]], rows[b], sems[b])

    def body(j, carry):
      for b in range(NBUF):
        ch = NBUF * j + b
        pltpu.make_async_copy(h_hbm.at[idx_s.at[ch]], rows[b], sems[b]).wait()
        pltpu.sync_copy(rows[b], acc_sh.at[idx_d.at[ch]], add=True)

        @pl.when(ch + NBUF < KB)
        def _():
          pltpu.async_copy(h_hbm.at[idx_s.at[ch + NBUF]], rows[b], sems[b])
      return carry

    lax.fori_loop(0, KB // NBUF, body, 0)
    for ch in range((KB // NBUF) * NBUF, KB):
      pltpu.make_async_copy(h_hbm.at[idx_s.at[ch]], rows[ch % NBUF],
                            sems[ch % NBUF]).wait()
      pltpu.sync_copy(rows[ch % NBUF], acc_sh.at[idx_d.at[ch]], add=True)

  plsc.subcore_barrier()

  def _flush(r, n):
    pltpu.sync_copy(acc_sh.at[pl.ds(r, n)], sums_out.at[c, pl.ds(r, n)])
    pltpu.sync_copy(z_rows.at[pl.ds(r, n)], acc_sh.at[pl.ds(r, n)])
  _striped(s, _flush)
  pltpu.sync_copy(ones_hbm, rows2)
  plsc.subcore_barrier()

  for blk in range(NB):
    pltpu.sync_copy(dst_hbm.at[wid, blk], idx_d)

    def dbody(j, carry):
      pltpu.sync_copy(rows2, acc_sh.at[idx_d.at[j]], add=True)
      return carry

    lax.fori_loop(0, KB, dbody, 0)

  plsc.subcore_barrier()
  _striped(s, lambda r, n: pltpu.sync_copy(acc_sh.at[pl.ds(r, n)],
                                           deg_out.at[c, pl.ds(r, n)]))


RB = 2000


def _dense_body(x_ref, sp_ref, dp_ref, w12_ref, w2t_ref, b12_ref,
                gamma_ref, beta_ref, out_ref, *, with_ln):
  x = x_ref[...]
  ssum = sp_ref[0] + sp_ref[1]
  deg = dp_ref[0, :, 0:1] + dp_ref[1, :, 0:1]
  mean = ssum / jnp.maximum(deg, 1.0)
  out = (jnp.dot(x, w12_ref[...], preferred_element_type=jnp.float32)
         + b12_ref[...]
         - jnp.dot(mean, w2t_ref[...], preferred_element_type=jnp.float32))
  out = jnp.where(deg > 0.0, out, x)
  if with_ln:
    h = jnp.maximum(out, 0.0)
    mu = jnp.mean(h, axis=1, keepdims=True)
    var = jnp.mean((h - mu) * (h - mu), axis=1, keepdims=True)
    out = (h - mu) * lax.rsqrt(var + 1e-5) * gamma_ref[...] + beta_ref[...]
  out_ref[...] = out


def _make_dense(with_ln):
  body = functools.partial(_dense_body, with_ln=with_ln)
  return pl.pallas_call(
      body,
      grid=(N // RB,),
      in_specs=[
          pl.BlockSpec((RB, D), lambda i: (i, 0)),
          pl.BlockSpec((NC, RB, D), lambda i: (0, i, 0)),
          pl.BlockSpec((NC, RB, D), lambda i: (0, i, 0)),
          pl.BlockSpec((D, D), lambda i: (0, 0)),
          pl.BlockSpec((D, D), lambda i: (0, 0)),
          pl.BlockSpec((1, D), lambda i: (0, 0)),
          pl.BlockSpec((1, D), lambda i: (0, 0)),
          pl.BlockSpec((1, D), lambda i: (0, 0)),
      ],
      out_specs=pl.BlockSpec((RB, D), lambda i: (i, 0)),
      out_shape=jax.ShapeDtypeStruct((N, D), jnp.float32),
  )


_dense_ln = _make_dense(True)
_dense_out = _make_dense(False)


def kernel(x, edge_index, W1_0, b1_0, W2_0, b2_0, gamma, beta,
           W1_1, b1_1, W2_1, b2_1):
  src = edge_index[0].astype(jnp.int32)
  dst = edge_index[1].astype(jnp.int32)
  src = jnp.concatenate([src, jnp.zeros((EPAD - E,), jnp.int32)])
  dst = jnp.concatenate([dst, jnp.full((EPAD - E,), N, jnp.int32)])
  src4 = src.reshape(NW, NB, KB, CHUNK)
  dst4 = dst.reshape(NW, NB, KB, CHUNK)

  z_rows = jnp.zeros((NA, D), jnp.float32)
  ones = jnp.ones((CHUNK, D), jnp.float32)

  w12_0 = (W1_0 + W2_0).T
  w2t_0 = W2_0.T
  b12_0 = (b1_0 + b2_0).reshape(1, D)
  w12_1 = (W1_1 + W2_1).T
  w2t_1 = W2_1.T
  b12_1 = (b1_1 + b2_1).reshape(1, D)
  gamma2 = gamma.reshape(1, D)
  beta2 = beta.reshape(1, D)

  sums0, degp = _sc_agg_deg(x, src4, dst4, z_rows, ones)
  h1 = _dense_ln(x, sums0, degp, w12_0, w2t_0, b12_0, gamma2, beta2)
  sums1 = _sc_agg(h1, src4, dst4, z_rows)
  out = _dense_out(h1, sums1, degp, w12_1, w2t_1, b12_1, gamma2, beta2)
  return out
